# baseline (speedup 1.0000x reference)
import functools

import jax
import jax.numpy as jnp
from jax import lax
from jax.experimental import pallas as pl
from jax.experimental.pallas import tpu as pltpu
from jax.experimental.pallas import tpu_sc as plsc

_NC = 2
_NS = 16
_NW = _NC * _NS

_CHUNK = 128
_GCH = 4
_SUPER = _GCH * _CHUNK
_NBUF = 2


@functools.partial(jax.jit, static_argnames=("b", "d"))
def _sc_gather(table, idx3d, *, b, d):
    per_w = b // _NW
    n_super = per_w // _SUPER
    n_blocks = n_super // _NBUF
    mesh = plsc.VectorSubcoreMesh(core_axis_name="c", subcore_axis_name="s")

    @functools.partial(
        pl.kernel,
        out_type=jax.ShapeDtypeStruct((b, d), jnp.float32),
        mesh=mesh,
        scratch_types=[
            pltpu.VMEM((n_super, _SUPER), jnp.int32),
            pltpu.VMEM((_NBUF, _SUPER, d), jnp.float32),
            pltpu.SemaphoreType.DMA((_NBUF,)),
            pltpu.SemaphoreType.DMA((_NBUF,)),
        ],
        compiler_params=pltpu.CompilerParams(use_tc_tiling_on_sc=False),
    )
    def k(table_hbm, idx_hbm, out_hbm, idx_v, rows_v, sem_g, sem_w):
        wid = lax.axis_index("s") * _NC + lax.axis_index("c")
        base = wid * per_w

        pltpu.sync_copy(idx_hbm.at[pl.ds(wid * n_super, n_super)], idx_v)

        def start_gather(g, slot):
            pltpu.async_copy(table_hbm.at[idx_v.at[g]], rows_v.at[slot],
                             sem_g.at[slot])

        def start_write(g, slot):
            off = base + g * _SUPER
            pltpu.async_copy(rows_v.at[slot], out_hbm.at[pl.ds(off, _SUPER)],
                             sem_w.at[slot])

        def wait_gather(g, slot):
            pltpu.make_async_copy(table_hbm.at[idx_v.at[g]], rows_v.at[slot],
                                  sem_g.at[slot]).wait()

        def wait_write(g, slot):
            off = base + g * _SUPER
            pltpu.make_async_copy(rows_v.at[slot],
                                  out_hbm.at[pl.ds(off, _SUPER)],
                                  sem_w.at[slot]).wait()

        for s in range(_NBUF):
            start_gather(s, s)
        for s in range(_NBUF):
            wait_gather(s, s)
            start_write(s, s)

        def block_body(j, carry):
            g0 = j * _NBUF
            for s in range(_NBUF):
                wait_write(g0 - _NBUF + s, s)
                start_gather(g0 + s, s)
            for s in range(_NBUF):
                wait_gather(g0 + s, s)
                start_write(g0 + s, s)
            return carry

        lax.fori_loop(1, n_blocks, block_body, 0)

        g0 = (n_blocks - 1) * _NBUF
        for s in range(_NBUF):
            wait_write(g0 + s, s)

    return k(table, idx3d)


def kernel(token_ids, W):
    shape = token_ids.shape
    d = W.shape[1]
    idx = token_ids.reshape(-1, _SUPER).astype(jnp.int32)
    out = _sc_gather(W, idx, b=token_ids.size, d=d)
    return out.reshape(*shape, d)


# speedup vs baseline: 1.0017x; 1.0017x over previous
import functools

import jax
import jax.numpy as jnp
from jax import lax
from jax.experimental import pallas as pl
from jax.experimental.pallas import tpu as pltpu
from jax.experimental.pallas import tpu_sc as plsc

_NC = 2
_NS = 16
_NW = _NC * _NS

_TB = 4
_NB = 2


@jax.jit
def _sc_gather(table, ids):
    t, p = ids.shape
    d = table.shape[1]
    tpw = t // _NW
    n_grp = tpw // _TB
    n_blocks = n_grp // _NB
    mesh = plsc.VectorSubcoreMesh(core_axis_name="c", subcore_axis_name="s")

    @functools.partial(
        pl.kernel,
        out_type=jax.ShapeDtypeStruct((t, p, d), jnp.float32),
        mesh=mesh,
        scratch_types=[
            pltpu.VMEM((tpw, p), jnp.int32),
            pltpu.VMEM((_NB, _TB, p, d), jnp.float32),
            pltpu.SemaphoreType.DMA((_NB, _TB)),
            pltpu.SemaphoreType.DMA((_NB,)),
        ],
        compiler_params=pltpu.CompilerParams(use_tc_tiling_on_sc=False),
    )
    def k(table_hbm, ids_hbm, out_hbm, idx_v, rows_v, sem_g, sem_w):
        wid = lax.axis_index("s") * _NC + lax.axis_index("c")
        tok0 = wid * tpw

        pltpu.sync_copy(ids_hbm.at[pl.ds(tok0, tpw)], idx_v)

        def start_gathers(g, slot):
            for j in range(_TB):
                pltpu.async_copy(table_hbm.at[idx_v.at[g * _TB + j]],
                                 rows_v.at[slot].at[j], sem_g.at[slot, j])

        def wait_gathers(g, slot):
            for j in range(_TB):
                pltpu.make_async_copy(table_hbm.at[idx_v.at[g * _TB + j]],
                                      rows_v.at[slot].at[j],
                                      sem_g.at[slot, j]).wait()

        def start_write(g, slot):
            pltpu.async_copy(rows_v.at[slot],
                             out_hbm.at[pl.ds(tok0 + g * _TB, _TB)],
                             sem_w.at[slot])

        def wait_write(g, slot):
            pltpu.make_async_copy(rows_v.at[slot],
                                  out_hbm.at[pl.ds(tok0 + g * _TB, _TB)],
                                  sem_w.at[slot]).wait()

        for s in range(_NB):
            start_gathers(s, s)
        for s in range(_NB):
            wait_gathers(s, s)
            start_write(s, s)

        def block_body(bidx, carry):
            g0 = bidx * _NB
            for s in range(_NB):
                wait_write(g0 - _NB + s, s)
                start_gathers(g0 + s, s)
            for s in range(_NB):
                wait_gathers(g0 + s, s)
                start_write(g0 + s, s)
            return carry

        lax.fori_loop(1, n_blocks, block_body, 0)

        g0 = (n_blocks - 1) * _NB
        for s in range(_NB):
            wait_write(g0 + s, s)

    return k(table, ids)


def kernel(token_ids, W):
    return _sc_gather(W, token_ids.astype(jnp.int32))


# speedup vs baseline: 1.0021x; 1.0004x over previous
import functools

import jax
import jax.numpy as jnp
from jax import lax
from jax.experimental import pallas as pl
from jax.experimental.pallas import tpu as pltpu
from jax.experimental.pallas import tpu_sc as plsc

_NC = 2
_NS = 16
_NW = _NC * _NS

_NB = 8
_CH = 128


@jax.jit
def _sc_gather(table, ids):
    n_chunks_total, ch = ids.shape
    n = n_chunks_total * ch
    d = table.shape[1]
    per = n // _NW
    n_chunks = per // ch
    n_blocks = n_chunks // _NB
    mesh = plsc.VectorSubcoreMesh(core_axis_name="c", subcore_axis_name="s")

    @functools.partial(
        pl.kernel,
        out_type=jax.ShapeDtypeStruct((n, d), jnp.float32),
        mesh=mesh,
        scratch_types=[
            pltpu.VMEM((n_chunks, ch), jnp.int32),
            pltpu.VMEM((_NB, ch, d), jnp.float32),
            pltpu.SemaphoreType.DMA((_NB,)),
            pltpu.SemaphoreType.DMA((_NB,)),
        ],
        compiler_params=pltpu.CompilerParams(use_tc_tiling_on_sc=False),
    )
    def k(table_hbm, ids_hbm, out_hbm, idx_v, rows_v, sem_g, sem_w):
        wid = lax.axis_index("s") * _NC + lax.axis_index("c")
        row0 = wid * per

        pltpu.sync_copy(ids_hbm.at[pl.ds(wid * n_chunks, n_chunks)], idx_v)

        def start_gather(c, slot):
            pltpu.async_copy(table_hbm.at[idx_v.at[c]], rows_v.at[slot],
                             sem_g.at[slot])

        def wait_gather(c, slot):
            pltpu.make_async_copy(table_hbm.at[idx_v.at[c]], rows_v.at[slot],
                                  sem_g.at[slot]).wait()

        def start_write(c, slot):
            pltpu.async_copy(rows_v.at[slot],
                             out_hbm.at[pl.ds(row0 + c * ch, ch)],
                             sem_w.at[slot])

        def wait_write(c, slot):
            pltpu.make_async_copy(rows_v.at[slot],
                                  out_hbm.at[pl.ds(row0 + c * ch, ch)],
                                  sem_w.at[slot]).wait()

        for s in range(_NB):
            start_gather(s, s)
        for s in range(_NB):
            wait_gather(s, s)
            start_write(s, s)

        def block_body(bidx, carry):
            c0 = bidx * _NB
            for s in range(_NB):
                wait_write(c0 - _NB + s, s)
                start_gather(c0 + s, s)
            for s in range(_NB):
                wait_gather(c0 + s, s)
                start_write(c0 + s, s)
            return carry

        lax.fori_loop(1, n_blocks, block_body, 0)

        c0 = (n_blocks - 1) * _NB
        for s in range(_NB):
            wait_write(c0 + s, s)

    return k(table, ids)


def kernel(token_ids, W):
    t, p = token_ids.shape
    ids = token_ids.astype(jnp.int32).reshape(-1, _CH)
    out = _sc_gather(W, ids)
    return out.reshape(t, p, W.shape[1])


# speedup vs baseline: 1.0023x; 1.0002x over previous
import functools

import jax
import jax.numpy as jnp
from jax import lax
from jax.experimental import pallas as pl
from jax.experimental.pallas import tpu as pltpu
from jax.experimental.pallas import tpu_sc as plsc

_NC = 2
_NS = 16
_NW = _NC * _NS

_NB = 4
_CH = 256


@jax.jit
def _sc_gather(table, ids):
    n_chunks_total, ch = ids.shape
    n = n_chunks_total * ch
    d = table.shape[1]
    per = n // _NW
    n_chunks = per // ch
    n_blocks = n_chunks // _NB
    mesh = plsc.VectorSubcoreMesh(core_axis_name="c", subcore_axis_name="s")

    @functools.partial(
        pl.kernel,
        out_type=jax.ShapeDtypeStruct((n, d), jnp.float32),
        mesh=mesh,
        scratch_types=[
            pltpu.VMEM((n_chunks, ch), jnp.int32),
            pltpu.VMEM((_NB, ch, d), jnp.float32),
            pltpu.SemaphoreType.DMA((_NB,)),
            pltpu.SemaphoreType.DMA((_NB,)),
        ],
        compiler_params=pltpu.CompilerParams(use_tc_tiling_on_sc=False),
    )
    def k(table_hbm, ids_hbm, out_hbm, idx_v, rows_v, sem_g, sem_w):
        wid = lax.axis_index("s") * _NC + lax.axis_index("c")
        row0 = wid * per

        pltpu.sync_copy(ids_hbm.at[pl.ds(wid * n_chunks, n_chunks)], idx_v)

        def start_gather(c, slot):
            pltpu.async_copy(table_hbm.at[idx_v.at[c]], rows_v.at[slot],
                             sem_g.at[slot])

        def wait_gather(c, slot):
            pltpu.make_async_copy(table_hbm.at[idx_v.at[c]], rows_v.at[slot],
                                  sem_g.at[slot]).wait()

        def start_write(c, slot):
            pltpu.async_copy(rows_v.at[slot],
                             out_hbm.at[pl.ds(row0 + c * ch, ch)],
                             sem_w.at[slot])

        def wait_write(c, slot):
            pltpu.make_async_copy(rows_v.at[slot],
                                  out_hbm.at[pl.ds(row0 + c * ch, ch)],
                                  sem_w.at[slot]).wait()

        for s in range(_NB):
            start_gather(s, s)
        for s in range(_NB):
            wait_gather(s, s)
            start_write(s, s)

        def block_body(bidx, carry):
            c0 = bidx * _NB
            for s in range(_NB):
                wait_write(c0 - _NB + s, s)
                start_gather(c0 + s, s)
            for s in range(_NB):
                wait_gather(c0 + s, s)
                start_write(c0 + s, s)
            return carry

        lax.fori_loop(1, n_blocks, block_body, 0)

        c0 = (n_blocks - 1) * _NB
        for s in range(_NB):
            wait_write(c0 + s, s)

    return k(table, ids)


def kernel(token_ids, W):
    t, p = token_ids.shape
    ids = token_ids.astype(jnp.int32).reshape(-1, _CH)
    out = _sc_gather(W, ids)
    return out.reshape(t, p, W.shape[1])
